# baseline (speedup 1.0000x reference)
import jax
import jax.numpy as jnp
from jax import lax
from jax.experimental import pallas as pl
from jax.experimental.pallas import tpu as pltpu

_BATCH = 64
_SEQ_LEN = 262144
_MAX_DROP_LENGTH = 2048
_WIN = _MAX_DROP_LENGTH + 128
_CW = 32768
_NCC = _SEQ_LEN // _CW
_NCHUNK = (_BATCH // 8) * _NCC
_NBUF = 6


def _drop_bounds(batch, seq_len):
    rkey = jax.random.key(42)
    k_start, k_len = jax.random.split(rkey)
    drop_start = jax.random.randint(k_start, (batch,), 0, seq_len // 2)
    drop_len = jax.random.randint(k_len, (batch,), 0, _MAX_DROP_LENGTH)
    drop_end = jnp.minimum(drop_start + drop_len, seq_len)
    return drop_start.astype(jnp.int32), drop_end.astype(jnp.int32)


def _body(s_ref, e_ref, ca_ref, s2d, e2d, ca2d, x_hbm, o_hbm,
          bufs, wins, sem_in, sem_out, sem_win, sem_wout):

    def chunk_in(i):
        g, c = divmod(i, _NCC)
        return pltpu.make_async_copy(
            x_hbm.at[pl.ds(8 * g, 8), pl.ds(c * _CW, _CW)],
            bufs.at[i % _NBUF], sem_in.at[i % _NBUF])

    def chunk_out(i):
        g, c = divmod(i, _NCC)
        return pltpu.make_async_copy(
            bufs.at[i % _NBUF],
            o_hbm.at[pl.ds(8 * g, 8), pl.ds(c * _CW, _CW)],
            sem_out.at[i % _NBUF])

    def win_in(r):
        ca = pl.multiple_of(ca_ref[r], 128)
        return pltpu.make_async_copy(
            x_hbm.at[r, pl.ds(ca, _WIN)], wins.at[r], sem_win.at[r])

    def win_out(r):
        ca = pl.multiple_of(ca_ref[r], 128)
        return pltpu.make_async_copy(
            wins.at[r], o_hbm.at[r, pl.ds(ca, _WIN)], sem_wout.at[r])

    for r in range(_BATCH):
        win_in(r).start()

    for i in range(min(_NBUF, _NCHUNK)):
        chunk_in(i).start()
    for i in range(_NCHUNK):
        chunk_in(i).wait()
        chunk_out(i).start()
        pre = i - _NBUF + 2
        if 0 <= pre and pre + _NBUF < _NCHUNK:
            chunk_out(pre).wait()
            chunk_in(pre + _NBUF).start()

    for r in range(_BATCH):
        win_in(r).wait()
    col = ca2d[:, 0:1] + lax.broadcasted_iota(jnp.int32, (_BATCH, _WIN), 1)
    mask = (col >= s2d[:, 0:1]) & (col < e2d[:, 0:1])
    wins[...] = jnp.where(mask, jnp.zeros((), wins.dtype), wins[...])

    for i in range(max(0, _NCHUNK - _NBUF), _NCHUNK):
        chunk_out(i).wait()

    for r in range(_BATCH):
        win_out(r).start()
    for r in range(_BATCH):
        win_out(r).wait()


def kernel(waveform):
    batch, seq_len = waveform.shape
    s, e = _drop_bounds(batch, seq_len)
    ca = (s // 128) * 128
    s2d = jnp.broadcast_to(s[:, None], (batch, 128))
    e2d = jnp.broadcast_to(e[:, None], (batch, 128))
    ca2d = jnp.broadcast_to(ca[:, None], (batch, 128))

    run = pl.pallas_call(
        _body,
        out_shape=jax.ShapeDtypeStruct((batch, seq_len), waveform.dtype),
        grid_spec=pltpu.PrefetchScalarGridSpec(
            num_scalar_prefetch=3,
            grid=(1,),
            in_specs=[
                pl.BlockSpec((batch, 128), lambda i, *_: (0, 0)),
                pl.BlockSpec((batch, 128), lambda i, *_: (0, 0)),
                pl.BlockSpec((batch, 128), lambda i, *_: (0, 0)),
                pl.BlockSpec(memory_space=pl.ANY),
            ],
            out_specs=pl.BlockSpec(memory_space=pl.ANY),
            scratch_shapes=[
                pltpu.VMEM((_NBUF, 8, _CW), jnp.float32),
                pltpu.VMEM((_BATCH, _WIN), jnp.float32),
                pltpu.SemaphoreType.DMA((_NBUF,)),
                pltpu.SemaphoreType.DMA((_NBUF,)),
                pltpu.SemaphoreType.DMA((_BATCH,)),
                pltpu.SemaphoreType.DMA((_BATCH,)),
            ],
        ),
    )
    return run(s, e, ca, s2d, e2d, ca2d, waveform)


# speedup vs baseline: 1.2086x; 1.2086x over previous
import jax
import jax.numpy as jnp
from jax import lax
from jax.experimental import pallas as pl
from jax.experimental.pallas import tpu as pltpu

_BATCH = 64
_SEQ_LEN = 262144
_MAX_DROP_LENGTH = 2048
_WIN = _MAX_DROP_LENGTH + 128


def _drop_bounds(batch, seq_len):
    rkey = jax.random.key(42)
    k_start, k_len = jax.random.split(rkey)
    drop_start = jax.random.randint(k_start, (batch,), 0, seq_len // 2)
    drop_len = jax.random.randint(k_len, (batch,), 0, _MAX_DROP_LENGTH)
    drop_end = jnp.minimum(drop_start + drop_len, seq_len)
    return drop_start.astype(jnp.int32), drop_end.astype(jnp.int32)


def _fix_kernel(s_ref, e_ref, ca_ref, s2d, e2d, ca2d, x_hbm, cp_any, o_hbm,
                wins, sem_in, sem_out):
    del cp_any

    def in_copy(r):
        ca = pl.multiple_of(ca_ref[r], 128)
        return pltpu.make_async_copy(
            x_hbm.at[r, pl.ds(ca, _WIN)], wins.at[r], sem_in.at[r])

    def out_copy(r):
        ca = pl.multiple_of(ca_ref[r], 128)
        return pltpu.make_async_copy(
            wins.at[r], o_hbm.at[r, pl.ds(ca, _WIN)], sem_out.at[r])

    for r in range(_BATCH):
        in_copy(r).start()
    for r in range(_BATCH):
        in_copy(r).wait()

    col = ca2d[:, 0:1] + lax.broadcasted_iota(jnp.int32, (_BATCH, _WIN), 1)
    mask = (col >= s2d[:, 0:1]) & (col < e2d[:, 0:1])
    wins[...] = jnp.where(mask, jnp.zeros((), wins.dtype), wins[...])

    for r in range(_BATCH):
        out_copy(r).start()
    for r in range(_BATCH):
        out_copy(r).wait()


def kernel(waveform):
    batch, seq_len = waveform.shape
    s, e = _drop_bounds(batch, seq_len)
    ca = (s // 128) * 128
    s2d = jnp.broadcast_to(s[:, None], (batch, 128))
    e2d = jnp.broadcast_to(e[:, None], (batch, 128))
    ca2d = jnp.broadcast_to(ca[:, None], (batch, 128))

    cp = jax.freeze(jax.new_ref(waveform))

    fix = pl.pallas_call(
        _fix_kernel,
        out_shape=jax.ShapeDtypeStruct((batch, seq_len), waveform.dtype),
        grid_spec=pltpu.PrefetchScalarGridSpec(
            num_scalar_prefetch=3,
            grid=(1,),
            in_specs=[
                pl.BlockSpec((batch, 128), lambda i, *_: (0, 0)),
                pl.BlockSpec((batch, 128), lambda i, *_: (0, 0)),
                pl.BlockSpec((batch, 128), lambda i, *_: (0, 0)),
                pl.BlockSpec(memory_space=pl.ANY),
                pl.BlockSpec(memory_space=pl.ANY),
            ],
            out_specs=pl.BlockSpec(memory_space=pl.ANY),
            scratch_shapes=[
                pltpu.VMEM((_BATCH, _WIN), jnp.float32),
                pltpu.SemaphoreType.DMA((_BATCH,)),
                pltpu.SemaphoreType.DMA((_BATCH,)),
            ],
        ),
        input_output_aliases={7: 0},
    )
    return fix(s, e, ca, s2d, e2d, ca2d, waveform, cp)
